# baseline (speedup 1.0000x reference)
import functools

import jax
import jax.numpy as jnp
from jax import lax
from jax.experimental import pallas as pl
from jax.experimental.pallas import tpu as pltpu
from jax.experimental.pallas import tpu_sc as plsc

_B, _C, _E, _P = 32, 32, 128, 256
_L = 16
_ROWS = 64
_NBUF = 6
_CHUNKS_PER_C = _E // _ROWS
_UNITS = _C * _CHUNKS_PER_C
_NGROUPS = _UNITS // _NBUF
_NTAIL = _UNITS - _NGROUPS * _NBUF


def _ppt_sc(X, perm_idx):
    mesh = plsc.VectorSubcoreMesh(core_axis_name="c", subcore_axis_name="s")

    @functools.partial(
        pl.kernel,
        out_type=jax.ShapeDtypeStruct((_B, _C, _E, _P), jnp.float32),
        mesh=mesh,
        compiler_params=pltpu.CompilerParams(
            needs_layout_passes=False, use_tc_tiling_on_sc=True),
        scratch_types=[
            pltpu.VMEM((_C, _P), jnp.int32),
            [pltpu.VMEM((_ROWS, _P), jnp.float32) for _ in range(_NBUF)],
            [pltpu.SemaphoreType.DMA for _ in range(_NBUF)],
            [pltpu.SemaphoreType.DMA for _ in range(_NBUF)],
        ],
    )
    def k(x_hbm, perm_hbm, out_hbm, perm_v, bufs, isems, osems):
        w = lax.axis_index("s") * 2 + lax.axis_index("c")
        pltpu.sync_copy(perm_hbm, perm_v)

        def start_in(u, kbuf):
            c, r0 = u // _CHUNKS_PER_C, (u % _CHUNKS_PER_C) * _ROWS
            pltpu.async_copy(
                x_hbm.at[w, c, pl.ds(r0, _ROWS)], bufs[kbuf], isems[kbuf])

        def wait_in(kbuf):
            pltpu.make_async_copy(
                x_hbm.at[0, 0, pl.ds(0, _ROWS)], bufs[kbuf], isems[kbuf]).wait()

        def start_out(u, kbuf):
            c, r0 = u // _CHUNKS_PER_C, (u % _CHUNKS_PER_C) * _ROWS
            pltpu.async_copy(
                bufs[kbuf], out_hbm.at[w, c, pl.ds(r0, _ROWS)], osems[kbuf])

        def wait_out(kbuf):
            pltpu.make_async_copy(
                bufs[kbuf], out_hbm.at[0, 0, pl.ds(0, _ROWS)], osems[kbuf]).wait()

        def permute_chunk(u, buf):
            c = u // _CHUNKS_PER_C
            cols = [perm_v[c, pl.ds(j * _L, _L)] for j in range(_P // _L)]
            rows0 = jnp.zeros((_L,), jnp.int32)

            @plsc.parallel_loop(0, _ROWS, unroll=4, carry=rows0)
            def _(r, rows):
                vals = [plsc.load_gather(buf, [rows, cols[j]])
                        for j in range(_P // _L)]
                for j in range(_P // _L):
                    buf[r, pl.ds(j * _L, _L)] = vals[j]
                return rows + 1

        for kbuf in range(_NBUF // 2):
            start_in(kbuf, kbuf)

        def do_group(g, carry):
            for ki in range(_NBUF):
                u = g * _NBUF + ki
                kj = (ki + _NBUF // 2) % _NBUF
                wait_in(ki)

                if ki < _NBUF // 2:
                    nxt = g * _NBUF + kj

                    @pl.when(g > 0)
                    def _():
                        wait_out(kj)
                        start_in(nxt, kj)

                    @pl.when(g == 0)
                    def _():
                        start_in(nxt, kj)
                else:
                    nxt = (g + 1) * _NBUF + kj

                    @pl.when(nxt < _UNITS)
                    def _():
                        wait_out(kj)
                        start_in(nxt, kj)

                permute_chunk(u, bufs[ki])
                start_out(u, ki)

            return carry

        lax.fori_loop(0, _NGROUPS, do_group, 0, unroll=False)

        for t in range(_NBUF // 2, _NTAIL):
            wait_out(t)
            start_in(_NGROUPS * _NBUF + t, t)
        for t in range(_NTAIL):
            u = _NGROUPS * _NBUF + t
            wait_in(t)
            permute_chunk(u, bufs[t])
            start_out(u, t)
        for kbuf in range(_NBUF):
            wait_out(kbuf)

    return k(X, perm_idx)


def kernel(X, perm_idx):
    return _ppt_sc(X, perm_idx)


# speedup vs baseline: 1.2261x; 1.2261x over previous
import functools

import jax
import jax.numpy as jnp
from jax import lax
from jax.experimental import pallas as pl
from jax.experimental.pallas import tpu as pltpu
from jax.experimental.pallas import tpu_sc as plsc

_B, _C, _E, _P = 32, 32, 128, 256
_L = 16
_ROWS = 64
_NBUF = 6
_CHUNKS_PER_C = _E // _ROWS
_UNITS = _C * _CHUNKS_PER_C
_NGROUPS = _UNITS // _NBUF
_NTAIL = _UNITS - _NGROUPS * _NBUF


def _ppt_sc(X, perm_idx):
    mesh = plsc.VectorSubcoreMesh(core_axis_name="c", subcore_axis_name="s")

    @functools.partial(
        pl.kernel,
        out_type=jax.ShapeDtypeStruct((_B, _C, _E, _P), jnp.float32),
        mesh=mesh,
        compiler_params=pltpu.CompilerParams(
            needs_layout_passes=False, use_tc_tiling_on_sc=True),
        scratch_types=[
            pltpu.VMEM((_C, _P), jnp.int32),
            [pltpu.VMEM((_ROWS, _P), jnp.float32) for _ in range(_NBUF)],
            [pltpu.SemaphoreType.DMA for _ in range(_NBUF)],
            [pltpu.SemaphoreType.DMA for _ in range(_NBUF)],
        ],
    )
    def k(x_hbm, perm_hbm, out_hbm, perm_v, bufs, isems, osems):
        w = lax.axis_index("s") * 2 + lax.axis_index("c")
        pltpu.sync_copy(perm_hbm, perm_v)

        def start_in(u, kbuf):
            c, r0 = u // _CHUNKS_PER_C, (u % _CHUNKS_PER_C) * _ROWS
            pltpu.async_copy(
                x_hbm.at[w, c, pl.ds(r0, _ROWS)], bufs[kbuf], isems[kbuf])

        def wait_in(kbuf):
            pltpu.make_async_copy(
                x_hbm.at[0, 0, pl.ds(0, _ROWS)], bufs[kbuf], isems[kbuf]).wait()

        def start_out(u, kbuf):
            c, r0 = u // _CHUNKS_PER_C, (u % _CHUNKS_PER_C) * _ROWS
            pltpu.async_copy(
                bufs[kbuf], out_hbm.at[w, c, pl.ds(r0, _ROWS)], osems[kbuf])

        def wait_out(kbuf):
            pltpu.make_async_copy(
                bufs[kbuf], out_hbm.at[0, 0, pl.ds(0, _ROWS)], osems[kbuf]).wait()

        def permute_chunk(u, buf):
            c = u // _CHUNKS_PER_C
            cols = [perm_v[c, pl.ds(j * _L, _L)] for j in range(_P // _L)]
            rows0 = jnp.zeros((_L,), jnp.int32)

            @plsc.parallel_loop(0, _ROWS, unroll=1, carry=rows0)
            def _(r, rows):
                vals = [plsc.load_gather(buf, [rows, cols[j]])
                        for j in range(_P // _L)]
                for j in range(_P // _L):
                    buf[r, pl.ds(j * _L, _L)] = vals[j]
                return rows + 1

        for kbuf in range(_NBUF // 2):
            start_in(kbuf, kbuf)

        def do_group(g, carry):
            for ki in range(_NBUF):
                u = g * _NBUF + ki
                kj = (ki + _NBUF // 2) % _NBUF
                wait_in(ki)

                if ki < _NBUF // 2:
                    nxt = g * _NBUF + kj

                    @pl.when(g > 0)
                    def _():
                        wait_out(kj)
                        start_in(nxt, kj)

                    @pl.when(g == 0)
                    def _():
                        start_in(nxt, kj)
                else:
                    nxt = (g + 1) * _NBUF + kj

                    @pl.when(nxt < _UNITS)
                    def _():
                        wait_out(kj)
                        start_in(nxt, kj)

                permute_chunk(u, bufs[ki])
                start_out(u, ki)

            return carry

        lax.fori_loop(0, _NGROUPS, do_group, 0, unroll=False)

        for t in range(_NBUF // 2, _NTAIL):
            wait_out(t)
            start_in(_NGROUPS * _NBUF + t, t)
        for t in range(_NTAIL):
            u = _NGROUPS * _NBUF + t
            wait_in(t)
            permute_chunk(u, bufs[t])
            start_out(u, t)
        for kbuf in range(_NBUF):
            wait_out(kbuf)

    return k(X, perm_idx)


def kernel(X, perm_idx):
    return _ppt_sc(X, perm_idx)


# speedup vs baseline: 1.2343x; 1.0067x over previous
import functools

import jax
import jax.numpy as jnp
from jax import lax
from jax.experimental import pallas as pl
from jax.experimental.pallas import tpu as pltpu
from jax.experimental.pallas import tpu_sc as plsc

_B, _C, _E, _P = 32, 32, 128, 256
_L = 16
_ROWS = 64
_NBUF = 7
_CHUNKS_PER_C = _E // _ROWS
_UNITS = _C * _CHUNKS_PER_C
_NGROUPS = _UNITS // _NBUF
_NTAIL = _UNITS - _NGROUPS * _NBUF


def _ppt_sc(X, perm_idx):
    mesh = plsc.VectorSubcoreMesh(core_axis_name="c", subcore_axis_name="s")

    @functools.partial(
        pl.kernel,
        out_type=jax.ShapeDtypeStruct((_B, _C, _E, _P), jnp.float32),
        mesh=mesh,
        compiler_params=pltpu.CompilerParams(
            needs_layout_passes=False, use_tc_tiling_on_sc=True),
        scratch_types=[
            pltpu.VMEM((_C, _P), jnp.int32),
            [pltpu.VMEM((_ROWS, _P), jnp.float32) for _ in range(_NBUF)],
            [pltpu.SemaphoreType.DMA for _ in range(_NBUF)],
            [pltpu.SemaphoreType.DMA for _ in range(_NBUF)],
        ],
    )
    def k(x_hbm, perm_hbm, out_hbm, perm_v, bufs, isems, osems):
        w = lax.axis_index("s") * 2 + lax.axis_index("c")
        pltpu.sync_copy(perm_hbm, perm_v)

        def start_in(u, kbuf):
            c, r0 = u // _CHUNKS_PER_C, (u % _CHUNKS_PER_C) * _ROWS
            pltpu.async_copy(
                x_hbm.at[w, c, pl.ds(r0, _ROWS)], bufs[kbuf], isems[kbuf])

        def wait_in(kbuf):
            pltpu.make_async_copy(
                x_hbm.at[0, 0, pl.ds(0, _ROWS)], bufs[kbuf], isems[kbuf]).wait()

        def start_out(u, kbuf):
            c, r0 = u // _CHUNKS_PER_C, (u % _CHUNKS_PER_C) * _ROWS
            pltpu.async_copy(
                bufs[kbuf], out_hbm.at[w, c, pl.ds(r0, _ROWS)], osems[kbuf])

        def wait_out(kbuf):
            pltpu.make_async_copy(
                bufs[kbuf], out_hbm.at[0, 0, pl.ds(0, _ROWS)], osems[kbuf]).wait()

        def permute_chunk(u, buf):
            c = u // _CHUNKS_PER_C
            cols = [perm_v[c, pl.ds(j * _L, _L)] for j in range(_P // _L)]
            rows0 = jnp.zeros((_L,), jnp.int32)

            @plsc.parallel_loop(0, _ROWS, unroll=1, carry=rows0)
            def _(r, rows):
                vals = [plsc.load_gather(buf, [rows, cols[j]])
                        for j in range(_P // _L)]
                for j in range(_P // _L):
                    buf[r, pl.ds(j * _L, _L)] = vals[j]
                return rows + 1

        for kbuf in range(_NBUF // 2):
            start_in(kbuf, kbuf)

        def do_group(g, carry):
            for ki in range(_NBUF):
                u = g * _NBUF + ki
                kj = (ki + _NBUF // 2) % _NBUF
                wait_in(ki)

                if kj > ki:
                    nxt = g * _NBUF + kj

                    @pl.when(g > 0)
                    def _():
                        wait_out(kj)
                        start_in(nxt, kj)

                    @pl.when(g == 0)
                    def _():
                        start_in(nxt, kj)
                else:
                    nxt = (g + 1) * _NBUF + kj

                    @pl.when(nxt < _UNITS)
                    def _():
                        wait_out(kj)
                        start_in(nxt, kj)

                permute_chunk(u, bufs[ki])
                start_out(u, ki)

            return carry

        lax.fori_loop(0, _NGROUPS, do_group, 0, unroll=False)

        for t in range(_NBUF // 2, _NTAIL):
            wait_out(t)
            start_in(_NGROUPS * _NBUF + t, t)
        for t in range(_NTAIL):
            u = _NGROUPS * _NBUF + t
            wait_in(t)
            permute_chunk(u, bufs[t])
            start_out(u, t)
        for kbuf in range(_NBUF):
            wait_out(kbuf)

    return k(X, perm_idx)


def kernel(X, perm_idx):
    return _ppt_sc(X, perm_idx)
